# Initial kernel scaffold; baseline (speedup 1.0000x reference)
import jax
import jax.numpy as jnp
from jax.experimental import pallas as pl


def kernel(x, token_table, pos_table):
    raise NotImplementedError("write your pallas kernel here")


# speedup vs baseline: 3.3487x; 3.3487x over previous
import functools

import jax
import jax.numpy as jnp
from jax import lax
from jax.experimental import pallas as pl
from jax.experimental.pallas import tpu as pltpu
from jax.experimental.pallas import tpu_sc as plsc

NUM_CORES = 2
NUM_SUBCORES = 16
NW = NUM_CORES * NUM_SUBCORES
LANES = 16

SEQ_PER_CHUNK = 2
GATHER_SPLIT = 5


def kernel(x, token_table, pos_table):
    B, L = x.shape
    V, D = token_table.shape
    N = B * L
    assert N % NW == 0
    rows_per_w = N // NW
    C = SEQ_PER_CHUNK * L
    assert rows_per_w % C == 0
    n_chunks = rows_per_w // C
    G = C // GATHER_SPLIT
    assert G <= 128 and C % GATHER_SPLIT == 0 and G % 8 == 0

    mesh = plsc.VectorSubcoreMesh(core_axis_name="c", subcore_axis_name="s")

    @functools.partial(
        pl.kernel,
        out_type=jax.ShapeDtypeStruct((N, D), jnp.float32),
        mesh=mesh,
        scratch_types=[
            pltpu.VMEM((C,), jnp.int32),
            pltpu.VMEM((C, D), jnp.float32),
            pltpu.VMEM((L, D), jnp.float32),
            pltpu.SemaphoreType.DMA,
        ],
        compiler_params=pltpu.CompilerParams(use_tc_tiling_on_sc=False),
    )
    def emb_kernel(x_hbm, tok_hbm, pos_hbm, out_hbm, idx_v, rows_v, pos_v, sem):
        wid = lax.axis_index("s") * NUM_CORES + lax.axis_index("c")
        base = wid * rows_per_w
        pltpu.sync_copy(pos_hbm, pos_v)

        def chunk_body(ci, carry):
            off = base + ci * C
            pltpu.sync_copy(x_hbm.at[pl.ds(off, C)], idx_v)
            copies = [
                pltpu.async_copy(
                    tok_hbm.at[idx_v.at[pl.ds(j * G, G)]],
                    rows_v.at[pl.ds(j * G, G)],
                    sem,
                )
                for j in range(GATHER_SPLIT)
            ]
            for c in copies:
                c.wait()

            def row_body(r, carry2):
                for s in range(SEQ_PER_CHUNK):
                    for d in range(D // LANES):
                        sl = pl.ds(d * LANES, LANES)
                        plsc.addupdate(rows_v.at[s * L + r, sl], pos_v[r, sl])
                return carry2

            lax.fori_loop(0, L, row_body, 0)
            pltpu.sync_copy(rows_v, out_hbm.at[pl.ds(off, C)])
            return carry

        lax.fori_loop(0, n_chunks, chunk_body, 0)

    out = emb_kernel(x.reshape(N).astype(jnp.int32), token_table, pos_table)
    return out.reshape(B, L, D)


# speedup vs baseline: 4.1363x; 1.2352x over previous
import functools

import jax
import jax.numpy as jnp
from jax import lax
from jax.experimental import pallas as pl
from jax.experimental.pallas import tpu as pltpu
from jax.experimental.pallas import tpu_sc as plsc

NUM_CORES = 2
NUM_SUBCORES = 16
NW = NUM_CORES * NUM_SUBCORES
LANES = 16

SEQ_PER_CHUNK = 2
GATHER_SPLIT = 5


def kernel(x, token_table, pos_table):
    B, L = x.shape
    V, D = token_table.shape
    N = B * L
    assert N % NW == 0
    rows_per_w = N // NW
    C = SEQ_PER_CHUNK * L
    assert rows_per_w % (2 * C) == 0
    n_chunks = rows_per_w // C
    G = C // GATHER_SPLIT
    assert G <= 128 and C % GATHER_SPLIT == 0 and G % 8 == 0
    DV = D // LANES

    mesh = plsc.VectorSubcoreMesh(core_axis_name="c", subcore_axis_name="s")

    @functools.partial(
        pl.kernel,
        out_type=jax.ShapeDtypeStruct((N, D), jnp.float32),
        mesh=mesh,
        scratch_types=[
            pltpu.VMEM((rows_per_w,), jnp.int32),
            pltpu.VMEM((C, D), jnp.float32),
            pltpu.VMEM((C, D), jnp.float32),
            pltpu.VMEM((L, D), jnp.float32),
            pltpu.SemaphoreType.DMA,
            pltpu.SemaphoreType.DMA,
            pltpu.SemaphoreType.DMA,
            pltpu.SemaphoreType.DMA,
        ],
        compiler_params=pltpu.CompilerParams(use_tc_tiling_on_sc=False),
    )
    def emb_kernel(x_hbm, tok_hbm, pos_hbm, out_hbm,
                   idx_v, rows0, rows1, pos_v, gsem0, gsem1, wsem0, wsem1):
        wid = lax.axis_index("s") * NUM_CORES + lax.axis_index("c")
        base = wid * rows_per_w
        rows = (rows0, rows1)
        gsem = (gsem0, gsem1)
        wsem = (wsem0, wsem1)

        pltpu.sync_copy(pos_hbm, pos_v)
        pltpu.sync_copy(x_hbm.at[pl.ds(base, rows_per_w)], idx_v)

        def fire_gathers(ci, b):
            for j in range(GATHER_SPLIT):
                pltpu.async_copy(
                    tok_hbm.at[idx_v.at[pl.ds(ci * C + j * G, G)]],
                    rows[b].at[pl.ds(j * G, G)],
                    gsem[b],
                )

        def wait_gathers(b):
            pltpu.make_async_copy(tok_hbm.at[pl.ds(0, C)], rows[b], gsem[b]).wait()

        def add_pos(b):
            @plsc.parallel_loop(0, L, unroll=4)
            def row_body(r):
                for d in range(DV):
                    sl = pl.ds(d * LANES, LANES)
                    pv = pos_v[r, sl]
                    for s in range(SEQ_PER_CHUNK):
                        plsc.addupdate(rows[b].at[s * L + r, sl], pv)

        def fire_writeback(ci, b):
            pltpu.async_copy(rows[b], out_hbm.at[pl.ds(base + ci * C, C)], wsem[b])

        def wait_writeback(b):
            pltpu.make_async_copy(rows[b], out_hbm.at[pl.ds(base, C)], wsem[b]).wait()

        fire_gathers(0, 0)
        fire_gathers(1, 1)

        def pair_body(p, carry):
            c0 = 2 * p
            wait_gathers(0)
            add_pos(0)
            fire_writeback(c0, 0)
            wait_gathers(1)
            add_pos(1)
            fire_writeback(c0 + 1, 1)

            @pl.when(c0 + 2 < n_chunks)
            def _():
                wait_writeback(0)
                fire_gathers(c0 + 2, 0)
                wait_writeback(1)
                fire_gathers(c0 + 3, 1)

            return carry

        lax.fori_loop(0, n_chunks // 2, pair_body, 0)
        wait_writeback(0)
        wait_writeback(1)

    out = emb_kernel(x.reshape(N).astype(jnp.int32), token_table, pos_table)
    return out.reshape(B, L, D)


# speedup vs baseline: 4.1405x; 1.0010x over previous
import functools

import jax
import jax.numpy as jnp
from jax import lax
from jax.experimental import pallas as pl
from jax.experimental.pallas import tpu as pltpu
from jax.experimental.pallas import tpu_sc as plsc

NUM_CORES = 2
NUM_SUBCORES = 16
NW = NUM_CORES * NUM_SUBCORES
LANES = 16

SEQ_PER_CHUNK = 2
GATHER_SPLIT = 5


def kernel(x, token_table, pos_table):
    B, L = x.shape
    V, D = token_table.shape
    N = B * L
    assert N % NW == 0
    rows_per_w = N // NW
    seqs_per_w = rows_per_w // L
    S = SEQ_PER_CHUNK
    C = S * L
    assert seqs_per_w % (2 * S) == 0
    n_chunks = seqs_per_w // S
    G = L // GATHER_SPLIT
    assert G <= 128 and L % GATHER_SPLIT == 0 and G % 8 == 0
    DV = D // LANES

    mesh = plsc.VectorSubcoreMesh(core_axis_name="c", subcore_axis_name="s")

    @functools.partial(
        pl.kernel,
        out_type=jax.ShapeDtypeStruct((B, L, D), jnp.float32),
        mesh=mesh,
        scratch_types=[
            pltpu.VMEM((rows_per_w,), jnp.int32),
            pltpu.VMEM((S, L, D), jnp.float32),
            pltpu.VMEM((S, L, D), jnp.float32),
            pltpu.VMEM((L, D), jnp.float32),
            pltpu.SemaphoreType.DMA,
            pltpu.SemaphoreType.DMA,
            pltpu.SemaphoreType.DMA,
            pltpu.SemaphoreType.DMA,
        ],
        compiler_params=pltpu.CompilerParams(use_tc_tiling_on_sc=False),
    )
    def emb_kernel(x_hbm, tok_hbm, pos_hbm, out_hbm,
                   idx_v, rows0, rows1, pos_v, gsem0, gsem1, wsem0, wsem1):
        wid = lax.axis_index("s") * NUM_CORES + lax.axis_index("c")
        base = wid * rows_per_w
        seq_base = wid * seqs_per_w
        rows = (rows0, rows1)
        gsem = (gsem0, gsem1)
        wsem = (wsem0, wsem1)

        pltpu.sync_copy(pos_hbm, pos_v)
        pltpu.sync_copy(x_hbm.at[pl.ds(base, rows_per_w)], idx_v)

        def fire_gathers(ci, b):
            for s in range(S):
                for j in range(GATHER_SPLIT):
                    pltpu.async_copy(
                        tok_hbm.at[idx_v.at[pl.ds(ci * C + s * L + j * G, G)]],
                        rows[b].at[s, pl.ds(j * G, G)],
                        gsem[b],
                    )

        def wait_gathers(b):
            pltpu.make_async_copy(
                out_hbm.at[pl.ds(0, S)], rows[b], gsem[b]
            ).wait()

        def add_pos(b):
            @plsc.parallel_loop(0, L, unroll=4)
            def row_body(r):
                for d in range(DV):
                    sl = pl.ds(d * LANES, LANES)
                    pv = pos_v[r, sl]
                    for s in range(S):
                        plsc.addupdate(rows[b].at[s, r, sl], pv)

        def fire_writeback(ci, b):
            pltpu.async_copy(
                rows[b], out_hbm.at[pl.ds(seq_base + ci * S, S)], wsem[b]
            )

        def wait_writeback(b):
            pltpu.make_async_copy(
                rows[b], out_hbm.at[pl.ds(seq_base, S)], wsem[b]
            ).wait()

        fire_gathers(0, 0)
        fire_gathers(1, 1)

        def pair_body(p, carry):
            c0 = 2 * p
            wait_gathers(0)
            add_pos(0)
            fire_writeback(c0, 0)
            wait_gathers(1)
            add_pos(1)
            fire_writeback(c0 + 1, 1)

            @pl.when(c0 + 2 < n_chunks)
            def _():
                wait_writeback(0)
                fire_gathers(c0 + 2, 0)
                wait_writeback(1)
                fire_gathers(c0 + 3, 1)

            return carry

        lax.fori_loop(0, n_chunks // 2, pair_body, 0)
        wait_writeback(0)
        wait_writeback(1)

    return emb_kernel(x.reshape(N).astype(jnp.int32), token_table, pos_table)


# speedup vs baseline: 30.3017x; 7.3184x over previous
import functools

import jax
import jax.numpy as jnp
from jax import lax
from jax.experimental import pallas as pl
from jax.experimental.pallas import tpu as pltpu
from jax.experimental.pallas import tpu_sc as plsc

NUM_CORES = 2
NUM_SUBCORES = 16
NW = NUM_CORES * NUM_SUBCORES
LANES = 16
SUB = 8
LANE = 128


def kernel(x, token_table, pos_table):
    B, L = x.shape
    V, D = token_table.shape
    BW = B // NW
    assert B % NW == 0 and BW == LANE and L % 2 == 0 and D % SUB == 0
    BG = BW // LANES
    DT = D // SUB

    mesh = plsc.VectorSubcoreMesh(core_axis_name="c", subcore_axis_name="s")

    @functools.partial(
        pl.kernel,
        out_type=jax.ShapeDtypeStruct((L, DT, NW, SUB, LANE), jnp.float32),
        mesh=mesh,
        scratch_types=[
            pltpu.VMEM((L, BW), jnp.int32),
            pltpu.VMEM((L, D), jnp.float32),
            pltpu.VMEM((BW, D), jnp.float32),
            pltpu.VMEM((BW, D), jnp.float32),
            pltpu.VMEM((DT, 1, SUB, LANE), jnp.float32),
            pltpu.VMEM((DT, 1, SUB, LANE), jnp.float32),
            pltpu.SemaphoreType.DMA,
            pltpu.SemaphoreType.DMA,
            pltpu.SemaphoreType.DMA,
            pltpu.SemaphoreType.DMA,
        ],
        compiler_params=pltpu.CompilerParams(
            use_tc_tiling_on_sc=False, needs_layout_passes=False
        ),
    )
    def emb_kernel(xt_hbm, tok_hbm, pos_hbm, out_hbm,
                   idx_v, pos_v, in0, in1, t0, t1, gsem0, gsem1, wsem0, wsem1):
        wid = lax.axis_index("s") * NUM_CORES + lax.axis_index("c")
        bbase = wid * BW
        inbuf = (in0, in1)
        tbuf = (t0, t1)
        gsem = (gsem0, gsem1)
        wsem = (wsem0, wsem1)

        pltpu.sync_copy(pos_hbm, pos_v)
        pltpu.sync_copy(xt_hbm.at[:, pl.ds(bbase, BW)], idx_v)

        def fire_gather(l, b):
            pltpu.async_copy(tok_hbm.at[idx_v.at[l]], inbuf[b], gsem[b])

        def wait_gather(b):
            pltpu.make_async_copy(tok_hbm.at[pl.ds(0, BW)], inbuf[b], gsem[b]).wait()

        def transpose_add(l, b):
            src = inbuf[b]
            dst = tbuf[b]
            rowbase = [
                lax.iota(jnp.int32, LANES) + jnp.int32(g * LANES) for g in range(BG)
            ]
            lrows = jnp.full((LANES,), l, jnp.int32)

            @plsc.parallel_loop(0, D, unroll=2)
            def d_body(d):
                cols = jnp.full((LANES,), d, jnp.int32)
                pv = plsc.load_gather(pos_v, [lrows, cols])
                dt = d // SUB
                dr = d % SUB
                for g in range(BG):
                    v = plsc.load_gather(src, [rowbase[g], cols])
                    dst[dt, 0, dr, pl.ds(g * LANES, LANES)] = v + pv

        def fire_writeback(l, b):
            pltpu.async_copy(
                tbuf[b], out_hbm.at[l, :, pl.ds(wid, 1)], wsem[b]
            )

        def wait_writeback(b):
            pltpu.make_async_copy(
                tbuf[b], out_hbm.at[0, :, pl.ds(wid, 1)], wsem[b]
            ).wait()

        fire_gather(0, 0)
        fire_gather(1, 1)

        def pair_body(p, carry):
            l0 = 2 * p
            wait_gather(0)
            transpose_add(l0, 0)
            fire_writeback(l0, 0)
            wait_gather(1)
            transpose_add(l0 + 1, 1)
            fire_writeback(l0 + 1, 1)

            @pl.when(l0 + 2 < L)
            def _():
                wait_writeback(0)
                fire_gather(l0 + 2, 0)
                wait_writeback(1)
                fire_gather(l0 + 3, 1)

            return carry

        lax.fori_loop(0, L // 2, pair_body, 0)
        wait_writeback(0)
        wait_writeback(1)

    out5 = emb_kernel(x.T.astype(jnp.int32), token_table, pos_table)
    out = out5.transpose(0, 1, 3, 2, 4).reshape(L, D, B)
    return jnp.transpose(out, (2, 0, 1))
